# Initial kernel scaffold; baseline (speedup 1.0000x reference)
import jax
import jax.numpy as jnp
from jax.experimental import pallas as pl


def kernel(x, W, b, keys):
    raise NotImplementedError("write your pallas kernel here")


# speedup vs baseline: 4.1201x; 4.1201x over previous
import jax
import jax.numpy as jnp
from jax.experimental import pallas as pl

N_TOKENS = 32768
INPUT_DIM = 768
QUERY_DIM = 128
K_EXPERTS = 64
TOP_K = 2

TILE = 512


def _gate_kernel(x_ref, w_ref, b_ref, keys_ref, probs_ref, idx_ref):
    query = jax.lax.dot_general(
        x_ref[...], w_ref[...], (((1,), (0,)), ((), ())),
        preferred_element_type=jnp.float32,
    ) + b_ref[...]
    logits = jax.lax.dot_general(
        query, keys_ref[...], (((1,), (1,)), ((), ())),
        preferred_element_type=jnp.float32,
    ) / jnp.sqrt(jnp.float32(QUERY_DIM))
    col = jax.lax.broadcasted_iota(jnp.int32, logits.shape, 1)
    big = jnp.int32(K_EXPERTS)
    l1 = jnp.max(logits, axis=1, keepdims=True)
    i1 = jnp.min(jnp.where(logits == l1, col, big), axis=1, keepdims=True)
    masked = jnp.where(col == i1, -jnp.inf, logits)
    l2 = jnp.max(masked, axis=1, keepdims=True)
    i2 = jnp.min(jnp.where(masked == l2, col, big), axis=1, keepdims=True)
    e2 = jnp.exp(l2 - l1)
    denom = jnp.float32(1.0) + e2
    p1 = jnp.float32(1.0) / denom
    p2 = e2 / denom
    zero = jnp.float32(0.0)
    probs_ref[...] = jnp.where(col == i1, p1, zero) + jnp.where(col == i2, p2, zero)
    idx_ref[...] = jnp.concatenate([i1, i2], axis=1)


@jax.jit
def kernel(x, W, b, keys):
    b2 = b.reshape(1, QUERY_DIM)
    n_tiles = N_TOKENS // TILE
    probs, idx = pl.pallas_call(
        _gate_kernel,
        grid=(n_tiles,),
        in_specs=[
            pl.BlockSpec((TILE, INPUT_DIM), lambda i: (i, 0)),
            pl.BlockSpec((INPUT_DIM, QUERY_DIM), lambda i: (0, 0)),
            pl.BlockSpec((1, QUERY_DIM), lambda i: (0, 0)),
            pl.BlockSpec((K_EXPERTS, QUERY_DIM), lambda i: (0, 0)),
        ],
        out_specs=(
            pl.BlockSpec((TILE, K_EXPERTS), lambda i: (i, 0)),
            pl.BlockSpec((TILE, TOP_K), lambda i: (i, 0)),
        ),
        out_shape=(
            jax.ShapeDtypeStruct((N_TOKENS, K_EXPERTS), jnp.float32),
            jax.ShapeDtypeStruct((N_TOKENS, TOP_K), jnp.int32),
        ),
    )(x, W, b2, keys)
    return (probs, idx)


# speedup vs baseline: 4.1342x; 1.0034x over previous
import jax
import jax.numpy as jnp
from jax.experimental import pallas as pl
from jax.experimental.pallas import tpu as pltpu

N_TOKENS = 32768
INPUT_DIM = 768
QUERY_DIM = 128
K_EXPERTS = 64
TOP_K = 2

TILE = 512


def _gate_kernel(x_ref, w_ref, b_ref, keys_ref, probs_ref, idx_ref):
    query = jax.lax.dot_general(
        x_ref[...], w_ref[...], (((1,), (0,)), ((), ())),
        preferred_element_type=jnp.float32,
    ) + b_ref[...]
    logits = jax.lax.dot_general(
        query, keys_ref[...], (((1,), (1,)), ((), ())),
        preferred_element_type=jnp.float32,
    ) / jnp.sqrt(jnp.float32(QUERY_DIM))
    col = jax.lax.broadcasted_iota(jnp.int32, logits.shape, 1)
    big = jnp.int32(K_EXPERTS)
    l1 = jnp.max(logits, axis=1, keepdims=True)
    i1 = jnp.min(jnp.where(logits == l1, col, big), axis=1, keepdims=True)
    masked = jnp.where(col == i1, -jnp.inf, logits)
    l2 = jnp.max(masked, axis=1, keepdims=True)
    i2 = jnp.min(jnp.where(masked == l2, col, big), axis=1, keepdims=True)
    e2 = jnp.exp(l2 - l1)
    denom = jnp.float32(1.0) + e2
    p1 = jnp.float32(1.0) / denom
    p2 = e2 / denom
    zero = jnp.float32(0.0)
    probs_ref[...] = jnp.where(col == i1, p1, zero) + jnp.where(col == i2, p2, zero)
    idx_ref[...] = jnp.concatenate([i1, i2], axis=1)


@jax.jit
def kernel(x, W, b, keys):
    b2 = b.reshape(1, QUERY_DIM)
    n_tiles = N_TOKENS // TILE
    probs, idx = pl.pallas_call(
        _gate_kernel,
        grid=(n_tiles,),
        in_specs=[
            pl.BlockSpec((TILE, INPUT_DIM), lambda i: (i, 0)),
            pl.BlockSpec((INPUT_DIM, QUERY_DIM), lambda i: (0, 0)),
            pl.BlockSpec((1, QUERY_DIM), lambda i: (0, 0)),
            pl.BlockSpec((K_EXPERTS, QUERY_DIM), lambda i: (0, 0)),
        ],
        out_specs=(
            pl.BlockSpec((TILE, K_EXPERTS), lambda i: (i, 0)),
            pl.BlockSpec((TILE, TOP_K), lambda i: (i, 0)),
        ),
        out_shape=(
            jax.ShapeDtypeStruct((N_TOKENS, K_EXPERTS), jnp.float32),
            jax.ShapeDtypeStruct((N_TOKENS, TOP_K), jnp.int32),
        ),
        compiler_params=pltpu.CompilerParams(
            dimension_semantics=("parallel",),
        ),
    )(x, W, b2, keys)
    return (probs, idx)


# speedup vs baseline: 5.3414x; 1.2920x over previous
import jax
import jax.numpy as jnp
from jax.experimental import pallas as pl
from jax.experimental.pallas import tpu as pltpu

N_TOKENS = 32768
INPUT_DIM = 768
QUERY_DIM = 128
K_EXPERTS = 64
TOP_K = 2

TILE = 1024


def _gate_kernel(x_ref, w_ref, b_ref, keys_ref, probs_ref, idx_ref):
    query = jax.lax.dot_general(
        x_ref[...], w_ref[...], (((1,), (0,)), ((), ())),
        preferred_element_type=jnp.float32,
    ) + b_ref[...]
    logits = jax.lax.dot_general(
        query, keys_ref[...], (((1,), (1,)), ((), ())),
        preferred_element_type=jnp.float32,
    ) / jnp.sqrt(jnp.float32(QUERY_DIM))
    col = jax.lax.broadcasted_iota(jnp.int32, logits.shape, 1)
    big = jnp.int32(K_EXPERTS)
    l1 = jnp.max(logits, axis=1, keepdims=True)
    i1 = jnp.min(jnp.where(logits == l1, col, big), axis=1, keepdims=True)
    masked = jnp.where(col == i1, -jnp.inf, logits)
    l2 = jnp.max(masked, axis=1, keepdims=True)
    i2 = jnp.min(jnp.where(masked == l2, col, big), axis=1, keepdims=True)
    e2 = jnp.exp(l2 - l1)
    denom = jnp.float32(1.0) + e2
    p1 = jnp.float32(1.0) / denom
    p2 = e2 / denom
    zero = jnp.float32(0.0)
    probs_ref[...] = jnp.where(col == i1, p1, zero) + jnp.where(col == i2, p2, zero)
    idx_ref[...] = jnp.concatenate([i1, i2], axis=1)


@jax.jit
def kernel(x, W, b, keys):
    b2 = b.reshape(1, QUERY_DIM)
    n_tiles = N_TOKENS // TILE
    probs, idx = pl.pallas_call(
        _gate_kernel,
        grid=(n_tiles,),
        in_specs=[
            pl.BlockSpec((TILE, INPUT_DIM), lambda i: (i, 0)),
            pl.BlockSpec((INPUT_DIM, QUERY_DIM), lambda i: (0, 0)),
            pl.BlockSpec((1, QUERY_DIM), lambda i: (0, 0)),
            pl.BlockSpec((K_EXPERTS, QUERY_DIM), lambda i: (0, 0)),
        ],
        out_specs=(
            pl.BlockSpec((TILE, K_EXPERTS), lambda i: (i, 0)),
            pl.BlockSpec((TILE, TOP_K), lambda i: (i, 0)),
        ),
        out_shape=(
            jax.ShapeDtypeStruct((N_TOKENS, K_EXPERTS), jnp.float32),
            jax.ShapeDtypeStruct((N_TOKENS, TOP_K), jnp.int32),
        ),
        compiler_params=pltpu.CompilerParams(
            dimension_semantics=("parallel",),
        ),
    )(x, W, b2, keys)
    return (probs, idx)


# speedup vs baseline: 6.0501x; 1.1327x over previous
import jax
import jax.numpy as jnp
from jax.experimental import pallas as pl
from jax.experimental.pallas import tpu as pltpu

N_TOKENS = 32768
INPUT_DIM = 768
QUERY_DIM = 128
K_EXPERTS = 64
TOP_K = 2

TILE = 2048


def _gate_kernel(x_ref, w_ref, b_ref, keys_ref, probs_ref, idx_ref):
    query = jax.lax.dot_general(
        x_ref[...], w_ref[...], (((1,), (0,)), ((), ())),
        preferred_element_type=jnp.float32,
    ) + b_ref[...]
    logits = jax.lax.dot_general(
        query, keys_ref[...], (((1,), (1,)), ((), ())),
        preferred_element_type=jnp.float32,
    ) / jnp.sqrt(jnp.float32(QUERY_DIM))
    col = jax.lax.broadcasted_iota(jnp.int32, logits.shape, 1)
    big = jnp.int32(K_EXPERTS)
    l1 = jnp.max(logits, axis=1, keepdims=True)
    i1 = jnp.min(jnp.where(logits == l1, col, big), axis=1, keepdims=True)
    masked = jnp.where(col == i1, -jnp.inf, logits)
    l2 = jnp.max(masked, axis=1, keepdims=True)
    i2 = jnp.min(jnp.where(masked == l2, col, big), axis=1, keepdims=True)
    e2 = jnp.exp(l2 - l1)
    denom = jnp.float32(1.0) + e2
    p1 = jnp.float32(1.0) / denom
    p2 = e2 / denom
    zero = jnp.float32(0.0)
    probs_ref[...] = jnp.where(col == i1, p1, zero) + jnp.where(col == i2, p2, zero)
    idx_ref[...] = jnp.concatenate([i1, i2], axis=1)


@jax.jit
def kernel(x, W, b, keys):
    b2 = b.reshape(1, QUERY_DIM)
    n_tiles = N_TOKENS // TILE
    probs, idx = pl.pallas_call(
        _gate_kernel,
        grid=(n_tiles,),
        in_specs=[
            pl.BlockSpec((TILE, INPUT_DIM), lambda i: (i, 0)),
            pl.BlockSpec((INPUT_DIM, QUERY_DIM), lambda i: (0, 0)),
            pl.BlockSpec((1, QUERY_DIM), lambda i: (0, 0)),
            pl.BlockSpec((K_EXPERTS, QUERY_DIM), lambda i: (0, 0)),
        ],
        out_specs=(
            pl.BlockSpec((TILE, K_EXPERTS), lambda i: (i, 0)),
            pl.BlockSpec((TILE, TOP_K), lambda i: (i, 0)),
        ),
        out_shape=(
            jax.ShapeDtypeStruct((N_TOKENS, K_EXPERTS), jnp.float32),
            jax.ShapeDtypeStruct((N_TOKENS, TOP_K), jnp.int32),
        ),
        compiler_params=pltpu.CompilerParams(
            dimension_semantics=("parallel",),
        ),
    )(x, W, b2, keys)
    return (probs, idx)


# speedup vs baseline: 6.3684x; 1.0526x over previous
import jax
import jax.numpy as jnp
from jax.experimental import pallas as pl
from jax.experimental.pallas import tpu as pltpu

N_TOKENS = 32768
INPUT_DIM = 768
QUERY_DIM = 128
K_EXPERTS = 64
TOP_K = 2

TILE = 4096


def _gate_kernel(x_ref, w_ref, b_ref, keys_ref, probs_ref, idx_ref):
    query = jax.lax.dot_general(
        x_ref[...], w_ref[...], (((1,), (0,)), ((), ())),
        preferred_element_type=jnp.float32,
    ) + b_ref[...]
    logits = jax.lax.dot_general(
        query, keys_ref[...], (((1,), (1,)), ((), ())),
        preferred_element_type=jnp.float32,
    ) / jnp.sqrt(jnp.float32(QUERY_DIM))
    col = jax.lax.broadcasted_iota(jnp.int32, logits.shape, 1)
    big = jnp.int32(K_EXPERTS)
    l1 = jnp.max(logits, axis=1, keepdims=True)
    i1 = jnp.min(jnp.where(logits == l1, col, big), axis=1, keepdims=True)
    masked = jnp.where(col == i1, -jnp.inf, logits)
    l2 = jnp.max(masked, axis=1, keepdims=True)
    i2 = jnp.min(jnp.where(masked == l2, col, big), axis=1, keepdims=True)
    e2 = jnp.exp(l2 - l1)
    denom = jnp.float32(1.0) + e2
    p1 = jnp.float32(1.0) / denom
    p2 = e2 / denom
    zero = jnp.float32(0.0)
    probs_ref[...] = jnp.where(col == i1, p1, zero) + jnp.where(col == i2, p2, zero)
    idx_ref[...] = jnp.concatenate([i1, i2], axis=1)


@jax.jit
def kernel(x, W, b, keys):
    b2 = b.reshape(1, QUERY_DIM)
    n_tiles = N_TOKENS // TILE
    probs, idx = pl.pallas_call(
        _gate_kernel,
        grid=(n_tiles,),
        in_specs=[
            pl.BlockSpec((TILE, INPUT_DIM), lambda i: (i, 0)),
            pl.BlockSpec((INPUT_DIM, QUERY_DIM), lambda i: (0, 0)),
            pl.BlockSpec((1, QUERY_DIM), lambda i: (0, 0)),
            pl.BlockSpec((K_EXPERTS, QUERY_DIM), lambda i: (0, 0)),
        ],
        out_specs=(
            pl.BlockSpec((TILE, K_EXPERTS), lambda i: (i, 0)),
            pl.BlockSpec((TILE, TOP_K), lambda i: (i, 0)),
        ),
        out_shape=(
            jax.ShapeDtypeStruct((N_TOKENS, K_EXPERTS), jnp.float32),
            jax.ShapeDtypeStruct((N_TOKENS, TOP_K), jnp.int32),
        ),
        compiler_params=pltpu.CompilerParams(
            dimension_semantics=("parallel",),
        ),
    )(x, W, b2, keys)
    return (probs, idx)


# speedup vs baseline: 6.7032x; 1.0526x over previous
import jax
import jax.numpy as jnp
from jax.experimental import pallas as pl
from jax.experimental.pallas import tpu as pltpu

N_TOKENS = 32768
INPUT_DIM = 768
QUERY_DIM = 128
K_EXPERTS = 64
TOP_K = 2

TILE = 4096


def _gate_kernel(x_ref, w_ref, b_ref, keys_ref, probs_ref, idx_ref):
    query = jax.lax.dot_general(
        x_ref[...], w_ref[...], (((1,), (0,)), ((), ())),
        preferred_element_type=jnp.float32,
    ) + b_ref[...]
    logits = jax.lax.dot_general(
        query, keys_ref[...], (((1,), (1,)), ((), ())),
        preferred_element_type=jnp.float32,
    ) / jnp.sqrt(jnp.float32(QUERY_DIM))
    colf = jax.lax.broadcasted_iota(jnp.int32, logits.shape, 1).astype(jnp.float32)
    big = jnp.float32(K_EXPERTS)
    l1 = jnp.max(logits, axis=1, keepdims=True)
    i1 = jnp.min(jnp.where(logits == l1, colf, big), axis=1, keepdims=True)
    masked = jnp.where(colf == i1, -jnp.inf, logits)
    l2 = jnp.max(masked, axis=1, keepdims=True)
    i2 = jnp.min(jnp.where(masked == l2, colf, big), axis=1, keepdims=True)
    e2 = jnp.exp(l2 - l1)
    denom = jnp.float32(1.0) + e2
    p1 = jnp.float32(1.0) / denom
    p2 = e2 / denom
    zero = jnp.float32(0.0)
    probs_ref[...] = jnp.where(colf == i1, p1, zero) + jnp.where(colf == i2, p2, zero)
    idx_ref[...] = jnp.concatenate([i1, i2], axis=1).astype(jnp.int32)


@jax.jit
def kernel(x, W, b, keys):
    b2 = b.reshape(1, QUERY_DIM)
    n_tiles = N_TOKENS // TILE
    probs, idx = pl.pallas_call(
        _gate_kernel,
        grid=(n_tiles,),
        in_specs=[
            pl.BlockSpec((TILE, INPUT_DIM), lambda i: (i, 0)),
            pl.BlockSpec((INPUT_DIM, QUERY_DIM), lambda i: (0, 0)),
            pl.BlockSpec((1, QUERY_DIM), lambda i: (0, 0)),
            pl.BlockSpec((K_EXPERTS, QUERY_DIM), lambda i: (0, 0)),
        ],
        out_specs=(
            pl.BlockSpec((TILE, K_EXPERTS), lambda i: (i, 0)),
            pl.BlockSpec((TILE, TOP_K), lambda i: (i, 0)),
        ),
        out_shape=(
            jax.ShapeDtypeStruct((N_TOKENS, K_EXPERTS), jnp.float32),
            jax.ShapeDtypeStruct((N_TOKENS, TOP_K), jnp.int32),
        ),
        compiler_params=pltpu.CompilerParams(
            dimension_semantics=("parallel",),
        ),
    )(x, W, b2, keys)
    return (probs, idx)
